# Initial kernel scaffold; baseline (speedup 1.0000x reference)
import jax
import jax.numpy as jnp
from jax.experimental import pallas as pl


def kernel(fea, edge_index, adj_values, mu_W, mu_b, sigma_W, sigma_b):
    raise NotImplementedError("write your pallas kernel here")


# speedup vs baseline: 3.4282x; 3.4282x over previous
import jax
import jax.numpy as jnp
from jax import lax
from jax.experimental import pallas as pl
from jax.experimental.pallas import tpu as pltpu
from jax.experimental.pallas import tpu_sc as plsc

N = 10000
E = 160000
D = 256
HALF = 128
NC = 2
NS = 16
CHUNK = 128
EPS = -(-E // NS)
NCHUNK = -(-EPS // CHUNK)
EPAD = NS * NCHUNK * CHUNK
NFULL = N // CHUNK
NREM = N - NFULL * CHUNK
LANES = 16


def _spmm_body(x_hbm, col_hbm, row_hbm, adj_hbm, out_hbm,
               colv, rowv, adjv, rows, acc, sem):
    c = lax.axis_index("c")
    s = lax.axis_index("s")

    pltpu.sync_copy(col_hbm.at[s], colv)
    pltpu.sync_copy(row_hbm.at[s], rowv)
    pltpu.sync_copy(adj_hbm.at[s], adjv)

    zero = jnp.zeros((LANES,), jnp.float32)

    def zb(e, _):
        for j in range(HALF // LANES):
            rows[e, pl.ds(j * LANES, LANES)] = zero
        return 0

    lax.fori_loop(0, CHUNK, zb, 0)
    for t in range(-(-(NFULL + 1) // NS)):
        cid = t * NS + s

        @pl.when(cid < NFULL)
        def _():
            pltpu.sync_copy(rows, acc.at[pl.ds(cid * CHUNK, CHUNK)])

        @pl.when(cid == NFULL)
        def _():
            pltpu.sync_copy(rows.at[pl.ds(0, NREM)],
                            acc.at[pl.ds(NFULL * CHUNK, NREM)])
    plsc.subcore_barrier()

    def chunk_body(k, _):
        pltpu.async_copy(x_hbm.at[c].at[colv.at[k]], rows, sem).wait()

        def scale(g, _):
            a16 = adjv[k, pl.ds(g * LANES, LANES)]
            for l in range(LANES):
                e = g * LANES + l
                for j in range(HALF // LANES):
                    sl = pl.ds(j * LANES, LANES)
                    rows[e, sl] = rows[e, sl] * a16[l]
            return 0

        lax.fori_loop(0, CHUNK // LANES, scale, 0)
        pltpu.sync_copy(rows, acc.at[rowv.at[k]], add=True)
        return 0

    lax.fori_loop(0, NCHUNK, chunk_body, 0)
    plsc.subcore_barrier()

    for t in range(-(-(NFULL + 1) // NS)):
        cid = t * NS + s

        @pl.when(cid < NFULL)
        def _():
            pltpu.sync_copy(acc.at[pl.ds(cid * CHUNK, CHUNK)],
                            out_hbm.at[c].at[pl.ds(cid * CHUNK, CHUNK)])

        @pl.when(cid == NFULL)
        def _():
            pltpu.sync_copy(acc.at[pl.ds(NFULL * CHUNK, NREM)],
                            out_hbm.at[c].at[pl.ds(NFULL * CHUNK, NREM)])


_spmm = pl.kernel(
    _spmm_body,
    out_type=jax.ShapeDtypeStruct((NC, N, HALF), jnp.float32),
    mesh=plsc.VectorSubcoreMesh(core_axis_name="c", subcore_axis_name="s"),
    scratch_types=[
        pltpu.VMEM((NCHUNK, CHUNK), jnp.int32),
        pltpu.VMEM((NCHUNK, CHUNK), jnp.int32),
        pltpu.VMEM((NCHUNK, CHUNK), jnp.float32),
        pltpu.VMEM((CHUNK, HALF), jnp.float32),
        pltpu.VMEM_SHARED((N, HALF), jnp.float32),
        pltpu.SemaphoreType.DMA,
    ],
)


BN = 2000


def _mm_body(x_ref, w_ref, b_ref, y_ref):
    w = w_ref[...]
    acc = jnp.dot(x_ref[0], w[:HALF], preferred_element_type=jnp.float32)
    acc = acc + jnp.dot(x_ref[1], w[HALF:], preferred_element_type=jnp.float32)
    acc = acc + b_ref[...]
    y_ref[0] = acc[:, :HALF]
    y_ref[1] = acc[:, HALF:]


_mm_bias = pl.pallas_call(
    _mm_body,
    grid=(N // BN,),
    in_specs=[
        pl.BlockSpec((NC, BN, HALF), lambda i: (0, i, 0)),
        pl.BlockSpec((D, D), lambda i: (0, 0)),
        pl.BlockSpec((1, D), lambda i: (0, 0)),
    ],
    out_specs=pl.BlockSpec((NC, BN, HALF), lambda i: (0, i, 0)),
    out_shape=jax.ShapeDtypeStruct((NC, N, HALF), jnp.float32),
)


def _comb_body(f_ref, l1_ref, l2_ref, x3_ref, w_ref, b_ref, o_ref):
    w = w_ref[...]
    l3 = jnp.dot(x3_ref[0], w[:HALF], preferred_element_type=jnp.float32)
    l3 = l3 + jnp.dot(x3_ref[1], w[HALF:], preferred_element_type=jnp.float32)
    l3 = l3 + b_ref[...]
    l1 = jnp.concatenate([l1_ref[0], l1_ref[1]], axis=1)
    l2 = jnp.concatenate([l2_ref[0], l2_ref[1]], axis=1)
    o_ref[...] = (f_ref[...] + l1 + l2 + l3) * 0.25


_combine = pl.pallas_call(
    _comb_body,
    grid=(N // BN,),
    in_specs=[
        pl.BlockSpec((BN, D), lambda i: (i, 0)),
        pl.BlockSpec((NC, BN, HALF), lambda i: (0, i, 0)),
        pl.BlockSpec((NC, BN, HALF), lambda i: (0, i, 0)),
        pl.BlockSpec((NC, BN, HALF), lambda i: (0, i, 0)),
        pl.BlockSpec((D, D), lambda i: (0, 0)),
        pl.BlockSpec((1, D), lambda i: (0, 0)),
    ],
    out_specs=pl.BlockSpec((BN, D), lambda i: (i, 0)),
    out_shape=jax.ShapeDtypeStruct((N, D), jnp.float32),
)


def kernel(fea, edge_index, adj_values, mu_W, mu_b, sigma_W, sigma_b):
    row = edge_index[0].astype(jnp.int32)
    col = edge_index[1].astype(jnp.int32)
    pad = EPAD - E
    col3 = jnp.pad(col, (0, pad)).reshape(NS, NCHUNK, CHUNK)
    row3 = jnp.pad(row, (0, pad)).reshape(NS, NCHUNK, CHUNK)
    adj3 = jnp.pad(adj_values, (0, pad)).reshape(NS, NCHUNK, CHUNK)
    fea2 = fea.reshape(N, NC, HALF).transpose(1, 0, 2)

    a1 = _spmm(fea2, col3, row3, adj3)
    l1m = _mm_bias(a1, mu_W[0], mu_b[0][None])
    l1s = _mm_bias(a1, sigma_W[0], sigma_b[0][None])
    a2m = _spmm(l1m, col3, row3, adj3)
    a2s = _spmm(l1s, col3, row3, adj3)
    l2m = _mm_bias(a2m, mu_W[1], mu_b[1][None])
    l2s = _mm_bias(a2s, sigma_W[1], sigma_b[1][None])
    a3m = _spmm(l2m, col3, row3, adj3)
    a3s = _spmm(l2s, col3, row3, adj3)
    mu = _combine(fea, l1m, l2m, a3m, mu_W[2], mu_b[2][None])
    sigma = _combine(fea, l1s, l2s, a3s, sigma_W[2], sigma_b[2][None])
    return mu, sigma
